# Initial kernel scaffold; baseline (speedup 1.0000x reference)
import jax
import jax.numpy as jnp
from jax.experimental import pallas as pl


def kernel(features, pos_emb_batch):
    raise NotImplementedError("write your pallas kernel here")


# speedup vs baseline: 9.7950x; 9.7950x over previous
import functools

import jax
import jax.numpy as jnp
from jax.experimental import pallas as pl

_NUM_CLUSTERS = 8
_HIGHEST = jax.lax.Precision.HIGHEST


def _dot_t_bf16(a, b):
    return jax.lax.dot_general(a.astype(jnp.bfloat16), b.astype(jnp.bfloat16),
                               (((1,), (1,)), ((), ())),
                               preferred_element_type=jnp.float32)


def _argext_first(vals, iota, extreme, fill):
    return jnp.min(jnp.where(vals == extreme, iota, fill))


def _row_of(mat, i, n):
    ri = jax.lax.broadcasted_iota(jnp.int32, (n, n), 0)
    return jnp.sum(jnp.where(ri == i, mat, 0.0), axis=0, keepdims=True)


def _fps_onehot(D, k, c):
    lane1 = jax.lax.broadcasted_iota(jnp.int32, (1, c), 1)
    crow = jax.lax.broadcasted_iota(jnp.int32, (k, c), 0)
    lanek = jax.lax.broadcasted_iota(jnp.int32, (k, c), 1)

    s = jnp.sum(D, axis=0, keepdims=True)
    start = _argext_first(s, lane1, jnp.max(s), c)
    sel = jnp.where((crow == 0) & (lanek == start), 1.0, 0.0)
    mind = _row_of(D, start, c)

    def body(j, carry):
        sel, mind = carry
        f = _argext_first(mind, lane1, jnp.max(mind), c)
        sel = sel + jnp.where((crow == j) & (lanek == f), 1.0, 0.0)
        mind = jnp.minimum(mind, _row_of(D, f, c))
        return sel, mind

    sel, _ = jax.lax.fori_loop(1, k, body, (sel, mind))
    return sel


def _self_dist(x, c):
    G = _dot_t_bf16(x, x)
    n = jnp.sum(x * x, axis=1, keepdims=True)
    n_row = jnp.transpose(n)
    return jnp.sqrt(jnp.maximum(n + n_row - 2.0 * G, 0.0))


def _cross_dist_t(a, b, m, n):
    an = jnp.sum(a * a, axis=1, keepdims=True)
    bn = jnp.sum(b * b, axis=1, keepdims=True)
    bn_row = jnp.transpose(bn)
    return jnp.sqrt(jnp.maximum(an + bn_row - 2.0 * _dot_t_bf16(a, b), 0.0))


def _stage_a(feat_ref, pos_ref, out_ref, *, c, k):
    f = feat_ref[0]
    p = pos_ref[0]

    D = _self_dist(f, c)
    sel = _fps_onehot(D, k, c)
    centers = jax.lax.dot_general(sel, p, (((1,), (0,)), ((), ())),
                                  precision=_HIGHEST)

    dT = _cross_dist_t(centers, p, k, c)
    crow = jax.lax.broadcasted_iota(jnp.int32, (k, c), 0)
    lane = jax.lax.broadcasted_iota(jnp.int32, (k, c), 1)
    mcol = jnp.min(dT, axis=0, keepdims=True)
    ta = jnp.min(jnp.where(dT == mcol, crow, k), axis=0, keepdims=True)
    onehot = jnp.where(crow == ta, 1.0, 0.0)
    counts = jnp.sum(onehot, axis=1, keepdims=True)
    sums = jax.lax.dot_general(onehot, p, (((1,), (0,)), ((), ())),
                               precision=_HIGHEST)
    out_ref[0] = jnp.where(lane == 3, counts, sums)


def _stage_b(pos_ref, stats_ref, out_ref, *, c, k, base, rem):
    p = pos_ref[:]
    st = jnp.sum(stats_ref[:], axis=0)
    crow = jax.lax.broadcasted_iota(jnp.int32, (k, c), 0)
    lane = jax.lax.broadcasted_iota(jnp.int32, (k, c), 1)

    counts = jnp.sum(jnp.where(lane == 3, st, 0.0), axis=1, keepdims=True)
    sums = jnp.where(lane < 3, st, 0.0)
    avg = jnp.where(counts > 0, sums / jnp.maximum(counts, 1.0), 0.0)

    Dp = _self_dist(p, c)
    sel = _fps_onehot(Dp, k, c)
    centers = jax.lax.dot_general(sel, p, (((1,), (0,)), ((), ())),
                                  precision=_HIGHEST)

    M = _cross_dist_t(centers, avg, k, k)
    lanek = jax.lax.broadcasted_iota(jnp.int32, (k, k), 1)
    mrow = jnp.min(M, axis=1, keepdims=True)
    matching = jnp.min(jnp.where(M == mrow, lanek, k), axis=1, keepdims=True)
    onehot_m = jnp.where(lanek == matching, 1.0, 0.0)
    matched = jax.lax.dot_general(onehot_m, avg, (((1,), (0,)), ((), ())),
                                  precision=_HIGHEST)
    nc = (1.0 - 0.2) * centers + 0.2 * matched

    dT = _cross_dist_t(nc, p, k, c)
    crow_k1 = jax.lax.broadcasted_iota(jnp.int32, (k, 1), 0)
    sizes = base + jnp.where(crow_k1 < rem, 1, 0)
    lane1 = jax.lax.broadcasted_iota(jnp.int32, (1, c), 1)

    def body(ch, carry):
        assign, cnt = carry
        col = jnp.sum(jnp.where(lane == ch, dT, 0.0), axis=1, keepdims=True)
        masked = jnp.where(cnt < sizes, col, jnp.inf)
        cl = _argext_first(masked, crow_k1, jnp.min(masked), k)
        assign = jnp.where(lane1 == ch, cl, assign)
        cnt = cnt + jnp.where(crow_k1 == cl, 1, 0)
        return assign, cnt

    assign0 = jnp.zeros((1, c), jnp.int32)
    cnt0 = jnp.zeros((k, 1), jnp.int32)
    assign, _ = jax.lax.fori_loop(0, c, body, (assign0, cnt0))
    out_ref[:] = assign


def kernel(features, pos_emb_batch):
    b, c = features.shape[0], features.shape[1]
    k = _NUM_CLUSTERS
    d = features.shape[2] * features.shape[3]
    ff = features.reshape(b, c, d)
    pos_pad = jnp.pad(pos_emb_batch, ((0, 0), (0, 0), (0, c - 3)))

    stats = pl.pallas_call(
        functools.partial(_stage_a, c=c, k=k),
        grid=(b,),
        in_specs=[
            pl.BlockSpec((1, c, d), lambda i: (i, 0, 0)),
            pl.BlockSpec((1, c, c), lambda i: (i, 0, 0)),
        ],
        out_specs=pl.BlockSpec((1, k, c), lambda i: (i, 0, 0)),
        out_shape=jax.ShapeDtypeStruct((b, k, c), jnp.float32),
    )(ff, pos_pad)

    assign = pl.pallas_call(
        functools.partial(_stage_b, c=c, k=k, base=c // k, rem=c % k),
        in_specs=[
            pl.BlockSpec((c, c), lambda: (0, 0)),
            pl.BlockSpec((b, k, c), lambda: (0, 0, 0)),
        ],
        out_specs=pl.BlockSpec((1, c), lambda: (0, 0)),
        out_shape=jax.ShapeDtypeStruct((1, c), jnp.int32),
    )(pos_pad[0], stats)

    return assign.reshape(c)


# speedup vs baseline: 12.6525x; 1.2917x over previous
import functools

import jax
import jax.numpy as jnp
from jax.experimental import pallas as pl
from jax.experimental.pallas import tpu as pltpu

_NUM_CLUSTERS = 8
_HIGHEST = jax.lax.Precision.HIGHEST


def _dot_t_bf16(a, b):
    return jax.lax.dot_general(a.astype(jnp.bfloat16), b.astype(jnp.bfloat16),
                               (((1,), (1,)), ((), ())),
                               preferred_element_type=jnp.float32)


def _dot_t_exact(a, b):
    return jax.lax.dot_general(a, b, (((1,), (0,)), ((), ())),
                               precision=_HIGHEST)


def _argext_first(vals, iota, extreme, fill):
    return jnp.min(jnp.where(vals == extreme, iota, fill))


def _row_of(mat, i, n):
    ri = jax.lax.broadcasted_iota(jnp.int32, (n, n), 0)
    return jnp.sum(jnp.where(ri == i, mat, 0.0), axis=0, keepdims=True)


def _fps_onehot(D, k, c):
    lane1 = jax.lax.broadcasted_iota(jnp.int32, (1, c), 1)
    crow = jax.lax.broadcasted_iota(jnp.int32, (k, c), 0)
    lanek = jax.lax.broadcasted_iota(jnp.int32, (k, c), 1)

    s = jnp.sum(D, axis=0, keepdims=True)
    start = _argext_first(s, lane1, jnp.max(s), c)
    sel = jnp.where((crow == 0) & (lanek == start), 1.0, 0.0)
    mind = _row_of(D, start, c)

    def body(j, carry):
        sel, mind = carry
        f = _argext_first(mind, lane1, jnp.max(mind), c)
        sel = sel + jnp.where((crow == j) & (lanek == f), 1.0, 0.0)
        mind = jnp.minimum(mind, _row_of(D, f, c))
        return sel, mind

    sel, _ = jax.lax.fori_loop(1, k, body, (sel, mind))
    return sel


def _self_dist(x, c):
    G = _dot_t_bf16(x, x)
    n = jnp.sum(x * x, axis=1, keepdims=True)
    return jnp.sqrt(jnp.maximum(n + jnp.transpose(n) - 2.0 * G, 0.0))


def _cross_dist_t(a, b, m, n):
    an = jnp.sum(a * a, axis=1, keepdims=True)
    bn = jnp.sum(b * b, axis=1, keepdims=True)
    return jnp.sqrt(jnp.maximum(an + jnp.transpose(bn) - 2.0 * _dot_t_bf16(a, b),
                                0.0))


def _fused(feat_ref, pos_ref, posT_ref, out_ref, d_scr, dm_scr,
           *, b, c, k, base, rem):
    pid = pl.program_id(0)

    @pl.when(pid < b)
    def _gram():
        f = feat_ref[0]
        n = jnp.sum(f * f, axis=1, keepdims=True)
        G = _dot_t_bf16(f, f)
        D = jnp.sqrt(jnp.maximum(n + jnp.transpose(n) - 2.0 * G, 0.0))
        d_scr[pl.ds(pid, 1)] = jnp.expand_dims(D, 0)

    @pl.when(pid == b)
    def _route():
        D_all = d_scr[:]
        pos_all = pos_ref[:]
        posT_all = posT_ref[:]
        lane2 = jax.lax.broadcasted_iota(jnp.int32, (b, c), 1)
        i3 = jax.lax.broadcasted_iota(jnp.int32, (b, c, c), 2)
        ik1 = jax.lax.broadcasted_iota(jnp.int32, (b, k, c), 1)

        def extract(M, fi):
            return jnp.sum(jnp.where(i3 == fi[:, :, None], M, 0.0), axis=2)

        def arg_rowmax(v):
            m = jnp.max(v, axis=1, keepdims=True)
            return jnp.min(jnp.where(v == m, lane2, c), axis=1, keepdims=True)

        s_all = jnp.sum(D_all, axis=2)
        fidx = arg_rowmax(s_all)
        mind = extract(D_all, fidx)
        centers = jnp.where(ik1 == 0, extract(posT_all, fidx)[:, None, :], 0.0)

        def body(j, carry):
            centers, mind = carry
            f = arg_rowmax(mind)
            centers = centers + jnp.where(
                ik1 == j, extract(posT_all, f)[:, None, :], 0.0)
            mind = jnp.minimum(mind, extract(D_all, f))
            return centers, mind

        centers, _ = jax.lax.fori_loop(1, k, body, (centers, mind))

        crow_kc = jax.lax.broadcasted_iota(jnp.int32, (k, c), 0)
        lane_kc = jax.lax.broadcasted_iota(jnp.int32, (k, c), 1)
        st = jnp.zeros((k, c), jnp.float32)
        cntv = jnp.zeros((k, 1), jnp.float32)
        for bi in range(b):
            cb = centers[bi]
            pb = pos_all[bi]
            dT = _cross_dist_t(cb, pb, k, c)
            mcol = jnp.min(dT, axis=0, keepdims=True)
            ta = jnp.min(jnp.where(dT == mcol, crow_kc, k), axis=0,
                         keepdims=True)
            oh = jnp.where(crow_kc == ta, 1.0, 0.0)
            st = st + _dot_t_exact(oh, pb)
            cntv = cntv + jnp.sum(oh, axis=1, keepdims=True)
        avg = jnp.where(cntv > 0,
                        jnp.where(lane_kc < 3, st, 0.0) / jnp.maximum(cntv, 1.0),
                        0.0)

        p = pos_all[0]
        sel = _fps_onehot(_self_dist(p, c), k, c)
        centers0 = _dot_t_exact(sel, p)
        M = _cross_dist_t(centers0, avg, k, k)
        lanekk = jax.lax.broadcasted_iota(jnp.int32, (k, k), 1)
        mrow = jnp.min(M, axis=1, keepdims=True)
        matching = jnp.min(jnp.where(M == mrow, lanekk, k), axis=1,
                           keepdims=True)
        onehot_m = jnp.where(lanekk == matching, 1.0, 0.0)
        matched = _dot_t_exact(onehot_m, avg)
        nc = (1.0 - 0.2) * centers0 + 0.2 * matched

        pn = jnp.sum(p * p, axis=1, keepdims=True)
        ncn = jnp.sum(nc * nc, axis=1, keepdims=True)
        Gpc = jax.lax.dot_general(p.astype(jnp.bfloat16),
                                  nc.astype(jnp.bfloat16),
                                  (((1,), (1,)), ((), ())),
                                  preferred_element_type=jnp.float32)
        dmat = jnp.sqrt(jnp.maximum(pn + jnp.transpose(ncn) - 2.0 * Gpc, 0.0))
        dm_scr[:] = jnp.pad(dmat, ((0, 0), (0, c - k)))

        lane_r = jax.lax.broadcasted_iota(jnp.int32, (1, c), 1)
        sizes = jnp.where(lane_r < k, base, 0) + jnp.where(lane_r < rem, 1, 0)

        def gbody(ch, carry):
            assign, cnt = carry
            row = dm_scr[pl.ds(ch, 1), :]
            masked = jnp.where(cnt < sizes, row, jnp.inf)
            mm = jnp.min(masked, axis=1, keepdims=True)
            cl = jnp.min(jnp.where(masked == mm, lane_r, c), axis=1,
                         keepdims=True)
            assign = jnp.where(lane_r == ch, cl, assign)
            cnt = cnt + jnp.where(lane_r == cl, 1, 0)
            return assign, cnt

        assign0 = jnp.zeros((1, c), jnp.int32)
        assign, _ = jax.lax.fori_loop(0, c, gbody, (assign0, assign0))
        out_ref[:] = assign


def kernel(features, pos_emb_batch):
    b, c = features.shape[0], features.shape[1]
    k = _NUM_CLUSTERS
    d = features.shape[2] * features.shape[3]
    ff = features.reshape(b, c, d)
    pos_pad = jnp.pad(pos_emb_batch, ((0, 0), (0, 0), (0, c - 3)))
    posT = jnp.swapaxes(pos_pad, 1, 2)

    assign = pl.pallas_call(
        functools.partial(_fused, b=b, c=c, k=k, base=c // k, rem=c % k),
        grid=(b + 1,),
        in_specs=[
            pl.BlockSpec((1, c, d), lambda i, _b=b: (jnp.minimum(i, _b - 1), 0, 0)),
            pl.BlockSpec((b, c, c), lambda i: (0, 0, 0)),
            pl.BlockSpec((b, c, c), lambda i: (0, 0, 0)),
        ],
        out_specs=pl.BlockSpec((1, c), lambda i: (0, 0)),
        out_shape=jax.ShapeDtypeStruct((1, c), jnp.int32),
        scratch_shapes=[
            pltpu.VMEM((b, c, c), jnp.float32),
            pltpu.VMEM((c, c), jnp.float32),
        ],
    )(ff, pos_pad, posT)

    return assign.reshape(c)


# speedup vs baseline: 13.2825x; 1.0498x over previous
import functools

import jax
import jax.numpy as jnp
from jax.experimental import pallas as pl
from jax.experimental.pallas import tpu as pltpu

_NUM_CLUSTERS = 8
_HIGHEST = jax.lax.Precision.HIGHEST


def _dot_t_bf16(a, b):
    return jax.lax.dot_general(a.astype(jnp.bfloat16), b.astype(jnp.bfloat16),
                               (((1,), (1,)), ((), ())),
                               preferred_element_type=jnp.float32)


def _dot_t_exact(a, b):
    return jax.lax.dot_general(a, b, (((1,), (0,)), ((), ())),
                               precision=_HIGHEST)


def _argext_first(vals, iota, extreme, fill):
    return jnp.min(jnp.where(vals == extreme, iota, fill))


def _row_of(mat, i, n):
    ri = jax.lax.broadcasted_iota(jnp.int32, (n, n), 0)
    return jnp.sum(jnp.where(ri == i, mat, 0.0), axis=0, keepdims=True)


def _fps_onehot(D, k, c):
    lane1 = jax.lax.broadcasted_iota(jnp.int32, (1, c), 1)
    crow = jax.lax.broadcasted_iota(jnp.int32, (k, c), 0)
    lanek = jax.lax.broadcasted_iota(jnp.int32, (k, c), 1)

    s = jnp.sum(D, axis=0, keepdims=True)
    start = _argext_first(s, lane1, jnp.max(s), c)
    sel = jnp.where((crow == 0) & (lanek == start), 1.0, 0.0)
    mind = _row_of(D, start, c)

    def body(j, carry):
        sel, mind = carry
        f = _argext_first(mind, lane1, jnp.max(mind), c)
        sel = sel + jnp.where((crow == j) & (lanek == f), 1.0, 0.0)
        mind = jnp.minimum(mind, _row_of(D, f, c))
        return sel, mind

    sel, _ = jax.lax.fori_loop(1, k, body, (sel, mind))
    return sel


def _self_dist(x, c):
    G = _dot_t_bf16(x, x)
    n = jnp.sum(x * x, axis=1, keepdims=True)
    return jnp.sqrt(jnp.maximum(n + jnp.transpose(n) - 2.0 * G, 0.0))


def _cross_dist_t(a, b, m, n):
    an = jnp.sum(a * a, axis=1, keepdims=True)
    bn = jnp.sum(b * b, axis=1, keepdims=True)
    return jnp.sqrt(jnp.maximum(an + jnp.transpose(bn) - 2.0 * _dot_t_bf16(a, b),
                                0.0))


def _fused(feat_ref, pos_ref, posT_ref, out_ref, d_scr,
           *, b, c, k, base, rem):
    pid = pl.program_id(0)

    @pl.when(pid < b)
    def _gram():
        f = feat_ref[0]
        n = jnp.sum(f * f, axis=1, keepdims=True)
        G = _dot_t_bf16(f, f)
        D = jnp.sqrt(jnp.maximum(n + jnp.transpose(n) - 2.0 * G, 0.0))
        d_scr[pl.ds(pid, 1)] = jnp.expand_dims(D, 0)

    @pl.when(pid == b)
    def _route():
        D_all = d_scr[:]
        pos_all = pos_ref[:]
        posT8 = posT_ref[:, :k, :]
        lane2 = jax.lax.broadcasted_iota(jnp.int32, (b, c), 1)
        i3 = jax.lax.broadcasted_iota(jnp.int32, (b, c, c), 2)
        i38 = jax.lax.broadcasted_iota(jnp.int32, (b, k, c), 2)
        ik1 = jax.lax.broadcasted_iota(jnp.int32, (b, k, c), 1)

        def extract(M, fi):
            return jnp.sum(jnp.where(i3 == fi[:, :, None], M, 0.0), axis=2)

        def extract_pos(fi):
            cj = jnp.sum(jnp.where(i38 == fi[:, :, None], posT8, 0.0), axis=2)
            return jnp.pad(cj, ((0, 0), (0, c - k)))

        def arg_rowmax(v):
            m = jnp.max(v, axis=1, keepdims=True)
            return jnp.min(jnp.where(v == m, lane2, c), axis=1, keepdims=True)

        s_all = jnp.sum(D_all, axis=2)
        fidx = arg_rowmax(s_all)
        mind = extract(D_all, fidx)
        centers = jnp.where(ik1 == 0, extract_pos(fidx)[:, None, :], 0.0)
        for j in range(1, k):
            f = arg_rowmax(mind)
            centers = centers + jnp.where(
                ik1 == j, extract_pos(f)[:, None, :], 0.0)
            if j < k - 1:
                mind = jnp.minimum(mind, extract(D_all, f))

        crow_kc = jax.lax.broadcasted_iota(jnp.int32, (k, c), 0)
        lane_kc = jax.lax.broadcasted_iota(jnp.int32, (k, c), 1)
        st = jnp.zeros((k, c), jnp.float32)
        cntv = jnp.zeros((k, 1), jnp.float32)
        for bi in range(b):
            cb = centers[bi]
            pb = pos_all[bi]
            dT = _cross_dist_t(cb, pb, k, c)
            mcol = jnp.min(dT, axis=0, keepdims=True)
            ta = jnp.min(jnp.where(dT == mcol, crow_kc, k), axis=0,
                         keepdims=True)
            oh = jnp.where(crow_kc == ta, 1.0, 0.0)
            st = st + _dot_t_exact(oh, pb)
            cntv = cntv + jnp.sum(oh, axis=1, keepdims=True)
        avg = jnp.where(cntv > 0,
                        jnp.where(lane_kc < 3, st, 0.0) / jnp.maximum(cntv, 1.0),
                        0.0)

        p = pos_all[0]
        sel = _fps_onehot(_self_dist(p, c), k, c)
        centers0 = _dot_t_exact(sel, p)
        M = _cross_dist_t(centers0, avg, k, k)
        lanekk = jax.lax.broadcasted_iota(jnp.int32, (k, k), 1)
        mrow = jnp.min(M, axis=1, keepdims=True)
        matching = jnp.min(jnp.where(M == mrow, lanekk, k), axis=1,
                           keepdims=True)
        onehot_m = jnp.where(lanekk == matching, 1.0, 0.0)
        matched = _dot_t_exact(onehot_m, avg)
        nc = (1.0 - 0.2) * centers0 + 0.2 * matched

        pn = jnp.sum(p * p, axis=1, keepdims=True)
        ncn = jnp.sum(nc * nc, axis=1, keepdims=True)
        Gpc = jax.lax.dot_general(p.astype(jnp.bfloat16),
                                  nc.astype(jnp.bfloat16),
                                  (((1,), (1,)), ((), ())),
                                  preferred_element_type=jnp.float32)
        dmat = jnp.sqrt(jnp.maximum(pn + jnp.transpose(ncn) - 2.0 * Gpc, 0.0))

        lane_c = jax.lax.broadcasted_iota(jnp.int32, (1, c), 1)
        lane_k = jax.lax.broadcasted_iota(jnp.int32, (1, k), 1)
        sizes = base + jnp.where(lane_k < rem, 1, 0)
        assign = jnp.zeros((1, c), jnp.int32)
        cnt = jnp.zeros((1, k), jnp.int32)
        for ch in range(c):
            row = dmat[ch:ch + 1, :]
            masked = jnp.where(cnt < sizes, row, jnp.inf)
            mm = jnp.min(masked, axis=1, keepdims=True)
            cl = jnp.min(jnp.where(masked == mm, lane_k, k), axis=1,
                         keepdims=True)
            assign = jnp.where(lane_c == ch, cl, assign)
            cnt = cnt + jnp.where(lane_k == cl, 1, 0)
        out_ref[:] = assign


def kernel(features, pos_emb_batch):
    b, c = features.shape[0], features.shape[1]
    k = _NUM_CLUSTERS
    d = features.shape[2] * features.shape[3]
    ff = features.reshape(b, c, d)
    pos_pad = jnp.pad(pos_emb_batch, ((0, 0), (0, 0), (0, c - 3)))
    posT = jnp.swapaxes(pos_pad, 1, 2)

    assign = pl.pallas_call(
        functools.partial(_fused, b=b, c=c, k=k, base=c // k, rem=c % k),
        grid=(b + 1,),
        in_specs=[
            pl.BlockSpec((1, c, d), lambda i, _b=b: (jnp.minimum(i, _b - 1), 0, 0)),
            pl.BlockSpec((b, c, c), lambda i: (0, 0, 0)),
            pl.BlockSpec((b, c, c), lambda i: (0, 0, 0)),
        ],
        out_specs=pl.BlockSpec((1, c), lambda i: (0, 0)),
        out_shape=jax.ShapeDtypeStruct((1, c), jnp.int32),
        scratch_shapes=[
            pltpu.VMEM((b, c, c), jnp.float32),
        ],
    )(ff, pos_pad, posT)

    return assign.reshape(c)


# speedup vs baseline: 14.2452x; 1.0725x over previous
import functools

import jax
import jax.numpy as jnp
from jax.experimental import pallas as pl
from jax.experimental.pallas import tpu as pltpu

_NUM_CLUSTERS = 8
_HIGHEST = jax.lax.Precision.HIGHEST


def _dot_t_bf16(a, b):
    return jax.lax.dot_general(a.astype(jnp.bfloat16), b.astype(jnp.bfloat16),
                               (((1,), (1,)), ((), ())),
                               preferred_element_type=jnp.float32)


def _dot_t_exact(a, b):
    return jax.lax.dot_general(a, b, (((1,), (0,)), ((), ())),
                               precision=_HIGHEST)


def _argext_first(vals, iota, extreme, fill):
    return jnp.min(jnp.where(vals == extreme, iota, fill))


def _row_of(mat, i, n):
    ri = jax.lax.broadcasted_iota(jnp.int32, (n, n), 0)
    return jnp.sum(jnp.where(ri == i, mat, 0.0), axis=0, keepdims=True)


def _fps_onehot(D, k, c):
    lane1 = jax.lax.broadcasted_iota(jnp.int32, (1, c), 1)
    crow = jax.lax.broadcasted_iota(jnp.int32, (k, c), 0)
    lanek = jax.lax.broadcasted_iota(jnp.int32, (k, c), 1)

    s = jnp.sum(D, axis=0, keepdims=True)
    start = _argext_first(s, lane1, jnp.max(s), c)
    sel = jnp.where((crow == 0) & (lanek == start), 1.0, 0.0)
    mind = _row_of(D, start, c)

    def body(j, carry):
        sel, mind = carry
        f = _argext_first(mind, lane1, jnp.max(mind), c)
        sel = sel + jnp.where((crow == j) & (lanek == f), 1.0, 0.0)
        mind = jnp.minimum(mind, _row_of(D, f, c))
        return sel, mind

    sel, _ = jax.lax.fori_loop(1, k, body, (sel, mind))
    return sel


def _self_dist(x, c):
    G = _dot_t_bf16(x, x)
    n = jnp.sum(x * x, axis=1, keepdims=True)
    return jnp.sqrt(jnp.maximum(n + jnp.transpose(n) - 2.0 * G, 0.0))


def _cross_dist_t(a, b, m, n):
    an = jnp.sum(a * a, axis=1, keepdims=True)
    bn = jnp.sum(b * b, axis=1, keepdims=True)
    return jnp.sqrt(jnp.maximum(an + jnp.transpose(bn) - 2.0 * _dot_t_bf16(a, b),
                                0.0))


def _fused(feat_ref, pos_ref, posT_ref, out_ref, d_scr,
           *, b, c, k, base, rem):
    pid = pl.program_id(0)

    @pl.when(pid < b // 2)
    def _gram():
        for bi in range(2):
            f = feat_ref[bi]
            n = jnp.sum(f * f, axis=1, keepdims=True)
            G = _dot_t_bf16(f, f)
            D = jnp.sqrt(jnp.maximum(n + jnp.transpose(n) - 2.0 * G, 0.0))
            d_scr[pl.ds(pid * 2 + bi, 1)] = jnp.expand_dims(D, 0)

    @pl.when(pid == b // 2)
    def _route():
        D_all = d_scr[:]
        pos_all = pos_ref[:]
        posT8 = posT_ref[:, :k, :]
        lane2 = jax.lax.broadcasted_iota(jnp.int32, (b, c), 1)
        i3 = jax.lax.broadcasted_iota(jnp.int32, (b, c, c), 2)
        i38 = jax.lax.broadcasted_iota(jnp.int32, (b, k, c), 2)
        ik1 = jax.lax.broadcasted_iota(jnp.int32, (b, k, c), 1)

        def extract(M, fi):
            return jnp.sum(jnp.where(i3 == fi[:, :, None], M, 0.0), axis=2)

        def extract_pos(fi):
            cj = jnp.sum(jnp.where(i38 == fi[:, :, None], posT8, 0.0), axis=2)
            return jnp.pad(cj, ((0, 0), (0, c - k)))

        def arg_rowmax(v):
            m = jnp.max(v, axis=1, keepdims=True)
            return jnp.min(jnp.where(v == m, lane2, c), axis=1, keepdims=True)

        s_all = jnp.sum(D_all, axis=2)
        fidx = arg_rowmax(s_all)
        mind = extract(D_all, fidx)
        centers = jnp.where(ik1 == 0, extract_pos(fidx)[:, None, :], 0.0)
        for j in range(1, k):
            f = arg_rowmax(mind)
            centers = centers + jnp.where(
                ik1 == j, extract_pos(f)[:, None, :], 0.0)
            if j < k - 1:
                mind = jnp.minimum(mind, extract(D_all, f))

        crow_kc = jax.lax.broadcasted_iota(jnp.int32, (k, c), 0)
        lane_kc = jax.lax.broadcasted_iota(jnp.int32, (k, c), 1)
        st = jnp.zeros((k, c), jnp.float32)
        cntv = jnp.zeros((k, 1), jnp.float32)
        for bi in range(b):
            cb = centers[bi]
            pb = pos_all[bi]
            dT = _cross_dist_t(cb, pb, k, c)
            mcol = jnp.min(dT, axis=0, keepdims=True)
            ta = jnp.min(jnp.where(dT == mcol, crow_kc, k), axis=0,
                         keepdims=True)
            oh = jnp.where(crow_kc == ta, 1.0, 0.0)
            st = st + _dot_t_exact(oh, pb)
            cntv = cntv + jnp.sum(oh, axis=1, keepdims=True)
        avg = jnp.where(cntv > 0,
                        jnp.where(lane_kc < 3, st, 0.0) / jnp.maximum(cntv, 1.0),
                        0.0)

        p = pos_all[0]
        sel = _fps_onehot(_self_dist(p, c), k, c)
        centers0 = _dot_t_exact(sel, p)
        M = _cross_dist_t(centers0, avg, k, k)
        lanekk = jax.lax.broadcasted_iota(jnp.int32, (k, k), 1)
        mrow = jnp.min(M, axis=1, keepdims=True)
        matching = jnp.min(jnp.where(M == mrow, lanekk, k), axis=1,
                           keepdims=True)
        onehot_m = jnp.where(lanekk == matching, 1.0, 0.0)
        matched = _dot_t_exact(onehot_m, avg)
        nc = (1.0 - 0.2) * centers0 + 0.2 * matched

        pn = jnp.sum(p * p, axis=1, keepdims=True)
        ncn = jnp.sum(nc * nc, axis=1, keepdims=True)
        Gpc = jax.lax.dot_general(p.astype(jnp.bfloat16),
                                  nc.astype(jnp.bfloat16),
                                  (((1,), (1,)), ((), ())),
                                  preferred_element_type=jnp.float32)
        dmat = jnp.sqrt(jnp.maximum(pn + jnp.transpose(ncn) - 2.0 * Gpc, 0.0))

        lane_c = jax.lax.broadcasted_iota(jnp.int32, (1, c), 1)
        lane_k = jax.lax.broadcasted_iota(jnp.int32, (1, k), 1)
        sizes = base + jnp.where(lane_k < rem, 1, 0)
        assign = jnp.zeros((1, c), jnp.int32)
        cnt = jnp.zeros((1, k), jnp.int32)
        for ch in range(c):
            row = dmat[ch:ch + 1, :]
            masked = jnp.where(cnt < sizes, row, jnp.inf)
            mm = jnp.min(masked, axis=1, keepdims=True)
            cl = jnp.min(jnp.where(masked == mm, lane_k, k), axis=1,
                         keepdims=True)
            assign = jnp.where(lane_c == ch, cl, assign)
            cnt = cnt + jnp.where(lane_k == cl, 1, 0)
        out_ref[:] = assign


def kernel(features, pos_emb_batch):
    b, c = features.shape[0], features.shape[1]
    k = _NUM_CLUSTERS
    d = features.shape[2] * features.shape[3]
    ff = features.reshape(b, c, d)
    pos_pad = jnp.pad(pos_emb_batch, ((0, 0), (0, 0), (0, c - 3)))
    posT = jnp.swapaxes(pos_pad, 1, 2)

    assign = pl.pallas_call(
        functools.partial(_fused, b=b, c=c, k=k, base=c // k, rem=c % k),
        grid=(b // 2 + 1,),
        in_specs=[
            pl.BlockSpec((2, c, d), lambda i, _b=b: (jnp.minimum(i, _b // 2 - 1), 0, 0)),
            pl.BlockSpec((b, c, c), lambda i: (0, 0, 0)),
            pl.BlockSpec((b, c, c), lambda i: (0, 0, 0)),
        ],
        out_specs=pl.BlockSpec((1, c), lambda i: (0, 0)),
        out_shape=jax.ShapeDtypeStruct((1, c), jnp.int32),
        scratch_shapes=[
            pltpu.VMEM((b, c, c), jnp.float32),
        ],
    )(ff, pos_pad, posT)

    return assign.reshape(c)


# speedup vs baseline: 14.6344x; 1.0273x over previous
import functools

import jax
import jax.numpy as jnp
from jax.experimental import pallas as pl
from jax.experimental.pallas import tpu as pltpu

_NUM_CLUSTERS = 8
_HIGHEST = jax.lax.Precision.HIGHEST


def _dot_t_bf16(a, b):
    return jax.lax.dot_general(a.astype(jnp.bfloat16), b.astype(jnp.bfloat16),
                               (((1,), (1,)), ((), ())),
                               preferred_element_type=jnp.float32)


def _dot_t_exact(a, b):
    return jax.lax.dot_general(a, b, (((1,), (0,)), ((), ())),
                               precision=_HIGHEST)


def _argext_first(vals, iota, extreme, fill):
    return jnp.min(jnp.where(vals == extreme, iota, fill))


def _row_of(mat, i, n):
    ri = jax.lax.broadcasted_iota(jnp.int32, (n, n), 0)
    return jnp.sum(jnp.where(ri == i, mat, 0.0), axis=0, keepdims=True)


def _fps_onehot(D, k, c):
    lane1 = jax.lax.broadcasted_iota(jnp.int32, (1, c), 1)
    crow = jax.lax.broadcasted_iota(jnp.int32, (k, c), 0)
    lanek = jax.lax.broadcasted_iota(jnp.int32, (k, c), 1)

    s = jnp.sum(D, axis=0, keepdims=True)
    start = _argext_first(s, lane1, jnp.max(s), c)
    sel = jnp.where((crow == 0) & (lanek == start), 1.0, 0.0)
    mind = _row_of(D, start, c)

    def body(j, carry):
        sel, mind = carry
        f = _argext_first(mind, lane1, jnp.max(mind), c)
        sel = sel + jnp.where((crow == j) & (lanek == f), 1.0, 0.0)
        mind = jnp.minimum(mind, _row_of(D, f, c))
        return sel, mind

    sel, _ = jax.lax.fori_loop(1, k, body, (sel, mind))
    return sel


def _self_dist(x, c):
    G = _dot_t_bf16(x, x)
    n = jnp.sum(x * x, axis=1, keepdims=True)
    return jnp.sqrt(jnp.maximum(n + jnp.transpose(n) - 2.0 * G, 0.0))


def _cross_dist_t(a, b, m, n):
    an = jnp.sum(a * a, axis=1, keepdims=True)
    bn = jnp.sum(b * b, axis=1, keepdims=True)
    return jnp.sqrt(jnp.maximum(an + jnp.transpose(bn) - 2.0 * _dot_t_bf16(a, b),
                                0.0))


def _fused(feat_ref, pos_ref, posT_ref, out_ref, d_scr,
           *, b, c, k, base, rem):
    pid = pl.program_id(0)

    @pl.when(pid < b // 4)
    def _gram():
        for bi in range(4):
            f = feat_ref[bi]
            n = jnp.sum(f * f, axis=1, keepdims=True)
            G = _dot_t_bf16(f, f)
            D = jnp.sqrt(jnp.maximum(n + jnp.transpose(n) - 2.0 * G, 0.0))
            d_scr[pl.ds(pid * 4 + bi, 1)] = jnp.expand_dims(D, 0)

    @pl.when(pid == b // 4)
    def _route():
        D_all = d_scr[:]
        pos_all = pos_ref[:]
        posT8 = posT_ref[:, :k, :]
        lane2 = jax.lax.broadcasted_iota(jnp.int32, (b, c), 1)
        i3 = jax.lax.broadcasted_iota(jnp.int32, (b, c, c), 2)
        i38 = jax.lax.broadcasted_iota(jnp.int32, (b, k, c), 2)
        ik1 = jax.lax.broadcasted_iota(jnp.int32, (b, k, c), 1)

        def extract(M, fi):
            return jnp.sum(jnp.where(i3 == fi[:, :, None], M, 0.0), axis=2)

        def extract_pos(fi):
            cj = jnp.sum(jnp.where(i38 == fi[:, :, None], posT8, 0.0), axis=2)
            return jnp.pad(cj, ((0, 0), (0, c - k)))

        def arg_rowmax(v):
            m = jnp.max(v, axis=1, keepdims=True)
            return jnp.min(jnp.where(v == m, lane2, c), axis=1, keepdims=True)

        s_all = jnp.sum(D_all, axis=2)
        fidx = arg_rowmax(s_all)
        mind = extract(D_all, fidx)
        centers = jnp.where(ik1 == 0, extract_pos(fidx)[:, None, :], 0.0)
        for j in range(1, k):
            f = arg_rowmax(mind)
            centers = centers + jnp.where(
                ik1 == j, extract_pos(f)[:, None, :], 0.0)
            if j < k - 1:
                mind = jnp.minimum(mind, extract(D_all, f))

        crow_kc = jax.lax.broadcasted_iota(jnp.int32, (k, c), 0)
        lane_kc = jax.lax.broadcasted_iota(jnp.int32, (k, c), 1)
        st = jnp.zeros((k, c), jnp.float32)
        cntv = jnp.zeros((k, 1), jnp.float32)
        for bi in range(b):
            cb = centers[bi]
            pb = pos_all[bi]
            dT = _cross_dist_t(cb, pb, k, c)
            mcol = jnp.min(dT, axis=0, keepdims=True)
            ta = jnp.min(jnp.where(dT == mcol, crow_kc, k), axis=0,
                         keepdims=True)
            oh = jnp.where(crow_kc == ta, 1.0, 0.0)
            st = st + _dot_t_exact(oh, pb)
            cntv = cntv + jnp.sum(oh, axis=1, keepdims=True)
        avg = jnp.where(cntv > 0,
                        jnp.where(lane_kc < 3, st, 0.0) / jnp.maximum(cntv, 1.0),
                        0.0)

        p = pos_all[0]
        sel = _fps_onehot(_self_dist(p, c), k, c)
        centers0 = _dot_t_exact(sel, p)
        M = _cross_dist_t(centers0, avg, k, k)
        lanekk = jax.lax.broadcasted_iota(jnp.int32, (k, k), 1)
        mrow = jnp.min(M, axis=1, keepdims=True)
        matching = jnp.min(jnp.where(M == mrow, lanekk, k), axis=1,
                           keepdims=True)
        onehot_m = jnp.where(lanekk == matching, 1.0, 0.0)
        matched = _dot_t_exact(onehot_m, avg)
        nc = (1.0 - 0.2) * centers0 + 0.2 * matched

        pn = jnp.sum(p * p, axis=1, keepdims=True)
        ncn = jnp.sum(nc * nc, axis=1, keepdims=True)
        Gpc = jax.lax.dot_general(p.astype(jnp.bfloat16),
                                  nc.astype(jnp.bfloat16),
                                  (((1,), (1,)), ((), ())),
                                  preferred_element_type=jnp.float32)
        dmat = jnp.sqrt(jnp.maximum(pn + jnp.transpose(ncn) - 2.0 * Gpc, 0.0))

        lane_c = jax.lax.broadcasted_iota(jnp.int32, (1, c), 1)
        lane_k = jax.lax.broadcasted_iota(jnp.int32, (1, k), 1)
        sizes = base + jnp.where(lane_k < rem, 1, 0)
        assign = jnp.zeros((1, c), jnp.int32)
        cnt = jnp.zeros((1, k), jnp.int32)
        for ch in range(c):
            row = dmat[ch:ch + 1, :]
            masked = jnp.where(cnt < sizes, row, jnp.inf)
            mm = jnp.min(masked, axis=1, keepdims=True)
            cl = jnp.min(jnp.where(masked == mm, lane_k, k), axis=1,
                         keepdims=True)
            assign = jnp.where(lane_c == ch, cl, assign)
            cnt = cnt + jnp.where(lane_k == cl, 1, 0)
        out_ref[:] = assign


def kernel(features, pos_emb_batch):
    b, c = features.shape[0], features.shape[1]
    k = _NUM_CLUSTERS
    d = features.shape[2] * features.shape[3]
    ff = features.reshape(b, c, d)
    pos_pad = jnp.pad(pos_emb_batch, ((0, 0), (0, 0), (0, c - 3)))
    posT = jnp.swapaxes(pos_pad, 1, 2)

    assign = pl.pallas_call(
        functools.partial(_fused, b=b, c=c, k=k, base=c // k, rem=c % k),
        grid=(b // 4 + 1,),
        in_specs=[
            pl.BlockSpec((4, c, d), lambda i, _b=b: (jnp.minimum(i, _b // 4 - 1), 0, 0)),
            pl.BlockSpec((b, c, c), lambda i: (0, 0, 0)),
            pl.BlockSpec((b, c, c), lambda i: (0, 0, 0)),
        ],
        out_specs=pl.BlockSpec((1, c), lambda i: (0, 0)),
        out_shape=jax.ShapeDtypeStruct((1, c), jnp.int32),
        scratch_shapes=[
            pltpu.VMEM((b, c, c), jnp.float32),
        ],
    )(ff, pos_pad, posT)

    return assign.reshape(c)
